# baseline (speedup 1.0000x reference)
import functools

import jax
import jax.numpy as jnp
from jax import lax
from jax.experimental import pallas as pl
from jax.experimental.pallas import tpu as pltpu
from jax.experimental.pallas import tpu_sc as plsc

N_TOK = 16384
D_IN = 2048
G_GRP = 8
E_PER_G = 8
E_TOT = G_GRP * E_PER_G
F_PAD = 72
BLK = 512

NC = 2
NS = 16
NW = NC * NS
TOK_W = N_TOK // NW
LANES = 16
CHUNKS = TOK_W // LANES


def _gemm_exp_block(x_ref, w_ref, e_ref):
    z = jnp.dot(x_ref[...], w_ref[...],
                preferred_element_type=jnp.float32,
                precision=jax.lax.Precision.DEFAULT)
    e_ref[...] = jnp.exp(z)


def _tc_exp_logits(x, wct):
    return pl.pallas_call(
        _gemm_exp_block,
        grid=(N_TOK // BLK,),
        in_specs=[
            pl.BlockSpec((BLK, D_IN), lambda i: (i, 0)),
            pl.BlockSpec((D_IN, F_PAD), lambda i: (0, 0)),
        ],
        out_specs=pl.BlockSpec((BLK, F_PAD), lambda i: (i, 0)),
        out_shape=jax.ShapeDtypeStruct((N_TOK, F_PAD), jnp.float32),
    )(x, wct)


_SC_MESH = plsc.VectorSubcoreMesh(core_axis_name="c", subcore_axis_name="s")


@functools.partial(
    pl.kernel,
    mesh=_SC_MESH,
    compiler_params=pltpu.CompilerParams(needs_layout_passes=False),
    out_type=jax.ShapeDtypeStruct((N_TOK * E_TOT,), jnp.float32),
    scratch_types=[
        pltpu.VMEM((TOK_W * F_PAD,), jnp.float32),
        pltpu.VMEM((TOK_W * E_TOT,), jnp.float32),
    ],
)
def _sc_router(e_hbm, nw_hbm, e_v, nw_v):
    wid = lax.axis_index("s") * NC + lax.axis_index("c")
    tok0 = wid * TOK_W
    pltpu.sync_copy(e_hbm.at[pl.ds(tok0 * F_PAD, TOK_W * F_PAD)], e_v)
    lane = lax.iota(jnp.int32, LANES)

    @plsc.parallel_loop(0, CHUNKS, 1, unroll=4)
    def chunk(t):
        rows = t * LANES + lane
        ebase = rows * F_PAD
        obase = rows * E_TOT

        def feat(j):
            return plsc.load_gather(e_v, [ebase + j])

        ge = [feat(E_TOT + g) for g in range(G_GRP)]
        gsum = ge[0]
        for g in range(1, G_GRP):
            gsum = gsum + ge[g]
        grec = 1.0 / gsum

        wvals = []
        wsum = jnp.zeros((LANES,), jnp.float32)
        for g in range(G_GRP):
            gp = ge[g] * grec
            mgp = jnp.where(gp >= 0.125, gp, 0.0)
            es = [feat(g * E_PER_G + k) for k in range(E_PER_G)]
            esum = es[0]
            for k in range(1, E_PER_G):
                esum = esum + es[k]
            erec = 1.0 / esum
            for k in range(E_PER_G):
                ep = es[k] * erec
                w = jnp.where(ep >= 0.125, mgp * ep, 0.0)
                wsum = wsum + w
                wvals.append(w)

        wrec = 1.0 / jnp.maximum(wsum, 1e-9)
        for j in range(E_TOT):
            plsc.store_scatter(nw_v, [obase + j], wvals[j] * wrec)

    pltpu.sync_copy(nw_v, nw_hbm.at[pl.ds(tok0 * E_TOT, TOK_W * E_TOT)])


@jax.jit
def kernel(x, Wg, We):
    wct = jnp.concatenate([We, Wg], axis=0).T
    e = _tc_exp_logits(x, wct)
    nw = _sc_router(e.reshape(N_TOK * F_PAD)).reshape(N_TOK, E_TOT)
    return nw > 0.0, nw


# speedup vs baseline: 1.0697x; 1.0697x over previous
import functools

import jax
import jax.numpy as jnp
from jax import lax
from jax.experimental import pallas as pl
from jax.experimental.pallas import tpu as pltpu
from jax.experimental.pallas import tpu_sc as plsc

N_TOK = 16384
D_IN = 2048
G_GRP = 8
E_PER_G = 8
E_TOT = G_GRP * E_PER_G
F_PAD = 72
BLK = 1024

NC = 2
NS = 16
NW = NC * NS
TOK_W = N_TOK // NW
LANES = 16
CHUNKS = TOK_W // LANES


def _gemm_exp_block(x_ref, w_ref, e_ref):
    z = jnp.dot(x_ref[...], w_ref[...],
                preferred_element_type=jnp.float32,
                precision=jax.lax.Precision.DEFAULT)
    e_ref[...] = jnp.exp(z)


def _tc_exp_logits(x, wct):
    return pl.pallas_call(
        _gemm_exp_block,
        grid=(N_TOK // BLK,),
        in_specs=[
            pl.BlockSpec((BLK, D_IN), lambda i: (i, 0)),
            pl.BlockSpec((D_IN, F_PAD), lambda i: (0, 0)),
        ],
        out_specs=pl.BlockSpec((BLK, F_PAD), lambda i: (i, 0)),
        out_shape=jax.ShapeDtypeStruct((N_TOK, F_PAD), jnp.float32),
    )(x, wct)


_SC_MESH = plsc.VectorSubcoreMesh(core_axis_name="c", subcore_axis_name="s")


@functools.partial(
    pl.kernel,
    mesh=_SC_MESH,
    compiler_params=pltpu.CompilerParams(needs_layout_passes=False),
    out_type=jax.ShapeDtypeStruct((N_TOK * E_TOT,), jnp.float32),
    scratch_types=[
        pltpu.VMEM((TOK_W * F_PAD,), jnp.float32),
        pltpu.VMEM((TOK_W * E_TOT,), jnp.float32),
    ],
)
def _sc_router(e_hbm, nw_hbm, e_v, nw_v):
    wid = lax.axis_index("s") * NC + lax.axis_index("c")
    tok0 = wid * TOK_W
    pltpu.sync_copy(e_hbm.at[pl.ds(tok0 * F_PAD, TOK_W * F_PAD)], e_v)
    lane = lax.iota(jnp.int32, LANES)

    @plsc.parallel_loop(0, CHUNKS, 1, unroll=4)
    def chunk(t):
        rows = t * LANES + lane
        ebase = rows * F_PAD
        obase = rows * E_TOT

        def feat(j):
            return plsc.load_gather(e_v, [ebase + j])

        ge = [feat(E_TOT + g) for g in range(G_GRP)]
        gsum = ge[0]
        for g in range(1, G_GRP):
            gsum = gsum + ge[g]
        grec = 1.0 / gsum

        wvals = []
        wsum = jnp.zeros((LANES,), jnp.float32)
        for g in range(G_GRP):
            gp = ge[g] * grec
            mgp = jnp.where(gp >= 0.125, gp, 0.0)
            es = [feat(g * E_PER_G + k) for k in range(E_PER_G)]
            esum = es[0]
            for k in range(1, E_PER_G):
                esum = esum + es[k]
            erec = 1.0 / esum
            for k in range(E_PER_G):
                ep = es[k] * erec
                w = jnp.where(ep >= 0.125, mgp * ep, 0.0)
                wsum = wsum + w
                wvals.append(w)

        wrec = 1.0 / jnp.maximum(wsum, 1e-9)
        for j in range(E_TOT):
            plsc.store_scatter(nw_v, [obase + j], wvals[j] * wrec)

    pltpu.sync_copy(nw_v, nw_hbm.at[pl.ds(tok0 * E_TOT, TOK_W * E_TOT)])


@jax.jit
def kernel(x, Wg, We):
    wct = jnp.concatenate([We, Wg], axis=0).T
    e = _tc_exp_logits(x, wct)
    nw = _sc_router(e.reshape(N_TOK * F_PAD)).reshape(N_TOK, E_TOT)
    return nw > 0.0, nw
